# baseline (speedup 1.0000x reference)
import functools

import jax
import jax.numpy as jnp
from jax import lax
from jax.experimental import pallas as pl
from jax.experimental.pallas import tpu as pltpu
from jax.experimental.pallas import tpu_sc as plsc

RATIO = 4


def _entropy_topk_body(attn_ref, gidx_ref, ent_s):
    h = pl.program_id(0)
    k = pl.program_id(1)
    nk = pl.num_programs(1)
    a = attn_ref[...]
    e = (jnp.exp(a) * a)[0]
    lb, s = e.shape
    acc = e[:, 0:128]
    for t in range(1, s // 128):
        acc = acc + e[:, t * 128:(t + 1) * 128]
    a3 = acc.reshape(lb, 16, 8)
    p = a3[:, 0, :]
    for t in range(1, 16):
        p = p + a3[:, t, :]
    t1 = p[:, 0:4] + p[:, 4:8]
    t2 = t1[:, 0:2] + t1[:, 2:4]
    v = t2[:, 0:1] + t2[:, 1:2]
    ent_s[pl.ds(k, 1), :] = v.reshape(1, lb)

    @pl.when(k == nk - 1)
    def _topk():
        v = ent_s[...]
        nck, lb = v.shape
        kk = gidx_ref.shape[-1]
        idx_acc = jnp.zeros((1, kk), jnp.int32)
        for ci in range(nck):
            vi_col = v[ci:ci + 1, :].reshape(lb, 1)
            cnt = jnp.zeros((lb, 1), jnp.int32)
            for cj in range(nck):
                vj = v[cj:cj + 1, :]
                gt = vj > vi_col
                if cj < ci:
                    m = gt | (vj == vi_col)
                elif cj == ci:
                    jj = lax.broadcasted_iota(jnp.int32, (lb, lb), 1)
                    ii = lax.broadcasted_iota(jnp.int32, (lb, lb), 0)
                    m = gt | ((vj == vi_col) & (jj < ii))
                else:
                    m = gt
                cnt = cnt + jnp.sum(m.astype(jnp.int32), axis=1, keepdims=True)
            rr = lax.broadcasted_iota(jnp.int32, (lb, kk), 1)
            onehot = cnt == rr
            gi = lax.broadcasted_iota(jnp.int32, (lb, kk), 0) + ci * lb
            idx_acc = idx_acc + jnp.sum(
                jnp.where(onehot, gi, 0), axis=0, keepdims=True)
        gidx_ref[...] = (idx_acc * 16 + h).reshape(gidx_ref.shape)


def _entropy_topk(attn3, num_output):
    nh, l, s = attn3.shape
    lb = 256
    nk = l // lb
    return pl.pallas_call(
        _entropy_topk_body,
        grid=(nh, nk),
        in_specs=[pl.BlockSpec((1, lb, s), lambda h, k: (h, k, 0))],
        out_specs=pl.BlockSpec((1, 1, num_output), lambda h, k: (h, 0, 0)),
        out_shape=jax.ShapeDtypeStruct((nh, 1, num_output), jnp.int32),
        scratch_shapes=[pltpu.VMEM((nk, lb), jnp.float32)],
    )(attn3)


def _sc_gather(xf, cf, gidx, num_rows):
    mesh = plsc.VectorSubcoreMesh(core_axis_name="c", subcore_axis_name="s")
    info = plsc.get_sparse_core_info()
    nw = info.num_cores * info.num_subcores
    rows_per_w = num_rows // nw
    cols_per_w = rows_per_w // 16

    @functools.partial(
        pl.kernel,
        out_type=(
            jax.ShapeDtypeStruct((num_rows, 64), jnp.float32),
            jax.ShapeDtypeStruct((num_rows, 64), jnp.float32),
        ),
        mesh=mesh,
        compiler_params=pltpu.CompilerParams(use_tc_tiling_on_sc=False),
        scratch_types=[
            pltpu.VMEM((rows_per_w,), jnp.int32),
            pltpu.VMEM((rows_per_w, 64), jnp.float32),
            pltpu.VMEM((rows_per_w, 64), jnp.float32),
            pltpu.SemaphoreType.DMA,
            pltpu.SemaphoreType.DMA,
            pltpu.SemaphoreType.DMA,
        ],
    )
    def gk(xf_hbm, cf_hbm, gidx_hbm, xo_hbm, co_hbm, tile_v, xr_v, cr_v,
           si, sx, sc):
        wid = lax.axis_index("s") * info.num_cores + lax.axis_index("c")
        icopies = []
        for h in range(16):
            icopies.append(pltpu.async_copy(
                gidx_hbm.at[pl.ds(h * 512 + wid * cols_per_w, cols_per_w)],
                tile_v.at[pl.ds(h * cols_per_w, cols_per_w)], si))
        for cp in icopies:
            cp.wait()
        copies = []
        for h in range(16):
            seg = tile_v.at[pl.ds(h * cols_per_w, cols_per_w)]
            copies.append(pltpu.async_copy(
                xf_hbm.at[seg], xr_v.at[pl.ds(h * cols_per_w, cols_per_w)],
                sx))
            copies.append(pltpu.async_copy(
                cf_hbm.at[seg], cr_v.at[pl.ds(h * cols_per_w, cols_per_w)],
                sc))
        for cp in copies:
            cp.wait()
        oiota = lax.iota(jnp.int32, 16) * 16 + wid * (cols_per_w * 16)
        ocopies = []
        for h in range(16):
            oidx = oiota + h
            ocopies.append(pltpu.async_copy(
                xr_v.at[pl.ds(h * cols_per_w, cols_per_w)], xo_hbm.at[oidx],
                sx))
            ocopies.append(pltpu.async_copy(
                cr_v.at[pl.ds(h * cols_per_w, cols_per_w)], co_hbm.at[oidx],
                sc))
        for cp in ocopies:
            cp.wait()

    return gk(xf, cf, gidx)


def kernel(x, coord, attn):
    b, l, c = x.shape
    nh, s = attn.shape[1], attn.shape[3]
    num_output = l // RATIO
    ch = c // nh

    gidx = _entropy_topk(attn.reshape(nh, l, s), num_output)
    gidx = gidx.reshape(nh * num_output)

    xf = x.reshape(l * nh, ch)
    cf = coord.reshape(l * nh, ch)
    xo, co = _sc_gather(xf, cf, gidx, num_output * nh)
    return (xo.reshape(b, num_output, c), co.reshape(b, num_output, c))


# speedup vs baseline: 2.1392x; 2.1392x over previous
import functools

import jax
import jax.numpy as jnp
from jax import lax
from jax.experimental import pallas as pl
from jax.experimental.pallas import tpu as pltpu
from jax.experimental.pallas import tpu_sc as plsc

RATIO = 4


def _entropy_topk_body(attn_ref, gidx_ref, ent_s):
    h = pl.program_id(0)
    k = pl.program_id(1)
    nk = pl.num_programs(1)
    a = attn_ref[...]
    e = (jnp.exp(a) * a)[0]
    lb, s = e.shape
    acc = e[:, 0:128]
    for t in range(1, s // 128):
        acc = acc + e[:, t * 128:(t + 1) * 128]
    tp = acc.T.reshape(16, 8, lb)
    p = tp[0]
    for t in range(1, 16):
        p = p + tp[t]
    t1 = p[0:4] + p[4:8]
    t2 = t1[0:2] + t1[2:4]
    v = t2[0:1] + t2[1:2]
    ent_s[pl.ds(k, 1), :] = v

    @pl.when(k == nk - 1)
    def _topk():
        v = ent_s[...]
        nck, lb = v.shape
        kk = gidx_ref.shape[-1]
        idx_acc = jnp.zeros((1, kk), jnp.float32)
        for ci in range(nck):
            vi_col = v[ci:ci + 1, :].reshape(lb, 1)
            cntmat = None
            for cj in range(nck):
                vj = v[cj:cj + 1, :]
                gt = vj > vi_col
                if cj < ci:
                    m = gt | (vj == vi_col)
                elif cj == ci:
                    jj = lax.broadcasted_iota(jnp.int32, (lb, lb), 1)
                    ii = lax.broadcasted_iota(jnp.int32, (lb, lb), 0)
                    m = gt | ((vj == vi_col) & (jj < ii))
                else:
                    m = gt
                mf = jnp.where(m, 1.0, 0.0)
                cntmat = mf if cntmat is None else cntmat + mf
            cnt = jnp.sum(cntmat, axis=1, keepdims=True)
            rr = lax.broadcasted_iota(jnp.int32, (lb, kk), 1).astype(
                jnp.float32)
            onehot = cnt == rr
            gi = lax.broadcasted_iota(jnp.int32, (lb, kk), 0).astype(
                jnp.float32) + ci * lb
            idx_acc = idx_acc + jnp.sum(
                jnp.where(onehot, gi, 0.0), axis=0, keepdims=True)
        gidx = idx_acc.astype(jnp.int32) * 16 + h
        gidx_ref[...] = gidx.reshape(gidx_ref.shape)


def _entropy_topk(attn3, num_output):
    nh, l, s = attn3.shape
    lb = 256
    nk = l // lb
    return pl.pallas_call(
        _entropy_topk_body,
        grid=(nh, nk),
        in_specs=[pl.BlockSpec((1, lb, s), lambda h, k: (h, k, 0))],
        out_specs=pl.BlockSpec((1, 1, num_output), lambda h, k: (h, 0, 0)),
        out_shape=jax.ShapeDtypeStruct((nh, 1, num_output), jnp.int32),
        scratch_shapes=[pltpu.VMEM((nk, lb), jnp.float32)],
    )(attn3)


def _sc_gather(xf, cf, gidx, num_rows):
    mesh = plsc.VectorSubcoreMesh(core_axis_name="c", subcore_axis_name="s")
    info = plsc.get_sparse_core_info()
    nw = info.num_cores * info.num_subcores
    rows_per_w = num_rows // nw
    cols_per_w = rows_per_w // 16

    @functools.partial(
        pl.kernel,
        out_type=(
            jax.ShapeDtypeStruct((num_rows, 64), jnp.float32),
            jax.ShapeDtypeStruct((num_rows, 64), jnp.float32),
        ),
        mesh=mesh,
        compiler_params=pltpu.CompilerParams(use_tc_tiling_on_sc=False),
        scratch_types=[
            pltpu.VMEM((rows_per_w,), jnp.int32),
            pltpu.VMEM((rows_per_w, 64), jnp.float32),
            pltpu.VMEM((rows_per_w, 64), jnp.float32),
            pltpu.SemaphoreType.DMA,
            pltpu.SemaphoreType.DMA,
            pltpu.SemaphoreType.DMA,
        ],
    )
    def gk(xf_hbm, cf_hbm, gidx_hbm, xo_hbm, co_hbm, tile_v, xr_v, cr_v,
           si, sx, sc):
        wid = lax.axis_index("s") * info.num_cores + lax.axis_index("c")
        icopies = []
        for h in range(16):
            icopies.append(pltpu.async_copy(
                gidx_hbm.at[pl.ds(h * 512 + wid * cols_per_w, cols_per_w)],
                tile_v.at[pl.ds(h * cols_per_w, cols_per_w)], si))
        for cp in icopies:
            cp.wait()
        copies = []
        for h in range(16):
            seg = tile_v.at[pl.ds(h * cols_per_w, cols_per_w)]
            copies.append(pltpu.async_copy(
                xf_hbm.at[seg], xr_v.at[pl.ds(h * cols_per_w, cols_per_w)],
                sx))
            copies.append(pltpu.async_copy(
                cf_hbm.at[seg], cr_v.at[pl.ds(h * cols_per_w, cols_per_w)],
                sc))
        for cp in copies:
            cp.wait()
        oiota = lax.iota(jnp.int32, 16) * 16 + wid * (cols_per_w * 16)
        ocopies = []
        for h in range(16):
            oidx = oiota + h
            ocopies.append(pltpu.async_copy(
                xr_v.at[pl.ds(h * cols_per_w, cols_per_w)], xo_hbm.at[oidx],
                sx))
            ocopies.append(pltpu.async_copy(
                cr_v.at[pl.ds(h * cols_per_w, cols_per_w)], co_hbm.at[oidx],
                sc))
        for cp in ocopies:
            cp.wait()

    return gk(xf, cf, gidx)


def kernel(x, coord, attn):
    b, l, c = x.shape
    nh, s = attn.shape[1], attn.shape[3]
    num_output = l // RATIO
    ch = c // nh

    gidx = _entropy_topk(attn.reshape(nh, l, s), num_output)
    gidx = gidx.reshape(nh * num_output)

    xf = x.reshape(l * nh, ch)
    cf = coord.reshape(l * nh, ch)
    xo, co = _sc_gather(xf, cf, gidx, num_output * nh)
    return (xo.reshape(b, num_output, c), co.reshape(b, num_output, c))
